# baseline (speedup 1.0000x reference)
import functools

import jax
import jax.numpy as jnp
from jax import lax
from jax.experimental import pallas as pl
from jax.experimental.pallas import tpu as pltpu
from jax.experimental.pallas import tpu_sc as plsc

NC = 2
NS = 16
NW = NC * NS


def _sc_gather(table, idx):
    V, D = table.shape
    B = idx.shape[0]
    b_per_w = B // NW
    CH = 128
    n_ch = b_per_w // CH
    mesh = plsc.VectorSubcoreMesh(core_axis_name="c", subcore_axis_name="s")

    @functools.partial(
        pl.kernel,
        mesh=mesh,
        out_type=jax.ShapeDtypeStruct((B, D), jnp.float32),
        scratch_types=[
            pltpu.VMEM((b_per_w,), jnp.int32),
            pltpu.VMEM((CH, D), jnp.float32),
            pltpu.SemaphoreType.DMA,
        ],
    )
    def gather_kernel(table_hbm, idx_hbm, out_hbm, idx_v, rows_v, sem):
        wid = lax.axis_index("s") * NC + lax.axis_index("c")
        base = wid * b_per_w
        pltpu.sync_copy(idx_hbm.at[pl.ds(base, b_per_w)], idx_v)

        @pl.loop(0, n_ch)
        def _(c):
            off = c * CH
            pltpu.async_copy(
                table_hbm.at[idx_v.at[pl.ds(off, CH)]], rows_v, sem
            ).wait()
            pltpu.sync_copy(rows_v, out_hbm.at[pl.ds(base + off, CH)])

    return gather_kernel(table, idx)


def _tc_project_chunk(emb, Wb, b2, position, B, chunk, n_chunks, dest):
    Bc, D = emb.shape
    M = Wb.shape[0]
    L = position.shape[0]
    BLK = 512
    j_blocks = L // BLK
    k_blocks = Bc // L
    blk_off = chunk * (Bc // BLK)

    def mm_kernel(*refs):
        emb_ref, w_ref, b_ref, pos_ref, out_ref = refs[-5:]
        acc = lax.dot_general(
            emb_ref[...].astype(jnp.bfloat16), w_ref[...],
            dimension_numbers=(((1,), (1,)), ((), ())),
            preferred_element_type=jnp.float32,
        )
        out_ref[...] = acc + b_ref[...] + pos_ref[...]

    data_specs = [
        pl.BlockSpec((BLK, D), lambda j, k: (k * j_blocks + j, 0)),
        pl.BlockSpec((M, D), lambda j, k: (0, 0)),
        pl.BlockSpec((1, M), lambda j, k: (0, 0)),
        pl.BlockSpec((BLK, M), lambda j, k: (j, 0)),
    ]
    if dest is None:
        in_specs, aliases, args = data_specs, {}, (emb, Wb, b2, position)
    else:
        in_specs = [pl.BlockSpec(memory_space=pl.ANY)] + data_specs
        aliases = {0: 0}
        args = (dest, emb, Wb, b2, position)

    return pl.pallas_call(
        mm_kernel,
        grid=(j_blocks, k_blocks),
        in_specs=in_specs,
        out_specs=pl.BlockSpec(
            (BLK, M), lambda j, k: (blk_off + k * j_blocks + j, 0)
        ),
        out_shape=jax.ShapeDtypeStruct((B, M), jnp.float32),
        input_output_aliases=aliases,
    )(*args)


def kernel(tokens, table, W, b, position):
    batch, seq = tokens.shape
    M = W.shape[0]
    B = batch * seq
    idx = tokens.reshape(-1).astype(jnp.int32)
    Wb = W.astype(jnp.bfloat16)
    b2 = b.reshape(1, M)

    n_chunks = 2
    Bc = B // n_chunks
    embs = [_sc_gather(table, lax.slice(idx, (c * Bc,), ((c + 1) * Bc,)))
            for c in range(n_chunks)]
    out = None
    for c in range(n_chunks):
        out = _tc_project_chunk(embs[c], Wb, b2, position, B, c, n_chunks, out)
    return out.reshape(batch, seq, M)


# speedup vs baseline: 1.1289x; 1.1289x over previous
import functools

import jax
import jax.numpy as jnp
from jax import lax
from jax.experimental import pallas as pl
from jax.experimental.pallas import tpu as pltpu
from jax.experimental.pallas import tpu_sc as plsc

NC = 2
NS = 16
NW = NC * NS


def _sc_gather(table, idx):
    V, D = table.shape
    B = idx.shape[0]
    b_per_w = B // NW
    CH = 128
    n_ch = b_per_w // CH
    mesh = plsc.VectorSubcoreMesh(core_axis_name="c", subcore_axis_name="s")

    @functools.partial(
        pl.kernel,
        mesh=mesh,
        out_type=jax.ShapeDtypeStruct((B, D), jnp.float32),
        scratch_types=[
            pltpu.VMEM((b_per_w,), jnp.int32),
            pltpu.VMEM((CH, D), jnp.float32),
            pltpu.SemaphoreType.DMA,
        ],
    )
    def gather_kernel(table_hbm, idx_hbm, out_hbm, idx_v, rows_v, sem):
        wid = lax.axis_index("s") * NC + lax.axis_index("c")
        base = wid * b_per_w
        pltpu.sync_copy(idx_hbm.at[pl.ds(base, b_per_w)], idx_v)

        @pl.loop(0, n_ch)
        def _(c):
            off = c * CH
            pltpu.async_copy(
                table_hbm.at[idx_v.at[pl.ds(off, CH)]], rows_v, sem
            ).wait()
            pltpu.sync_copy(rows_v, out_hbm.at[pl.ds(base + off, CH)])

    return gather_kernel(table, idx)


def _tc_project_chunk(emb, Wb, b2, position, B, chunk, n_chunks, dest):
    Bc, D = emb.shape
    M = Wb.shape[0]
    L = position.shape[0]
    BLK = 1024
    j_blocks = L // BLK
    k_blocks = Bc // L
    blk_off = chunk * (Bc // BLK)

    def mm_kernel(*refs):
        emb_ref, w_ref, b_ref, pos_ref, out_ref = refs[-5:]
        acc = lax.dot_general(
            emb_ref[...].astype(jnp.bfloat16), w_ref[...],
            dimension_numbers=(((1,), (1,)), ((), ())),
            preferred_element_type=jnp.float32,
        )
        out_ref[...] = acc + b_ref[...] + pos_ref[...]

    data_specs = [
        pl.BlockSpec((BLK, D), lambda j, k: (k * j_blocks + j, 0)),
        pl.BlockSpec((M, D), lambda j, k: (0, 0)),
        pl.BlockSpec((1, M), lambda j, k: (0, 0)),
        pl.BlockSpec((BLK, M), lambda j, k: (j, 0)),
    ]
    if dest is None:
        in_specs, aliases, args = data_specs, {}, (emb, Wb, b2, position)
    else:
        in_specs = [pl.BlockSpec(memory_space=pl.ANY)] + data_specs
        aliases = {0: 0}
        args = (dest, emb, Wb, b2, position)

    return pl.pallas_call(
        mm_kernel,
        grid=(j_blocks, k_blocks),
        in_specs=in_specs,
        out_specs=pl.BlockSpec(
            (BLK, M), lambda j, k: (blk_off + k * j_blocks + j, 0)
        ),
        out_shape=jax.ShapeDtypeStruct((B, M), jnp.float32),
        input_output_aliases=aliases,
    )(*args)


def kernel(tokens, table, W, b, position):
    batch, seq = tokens.shape
    M = W.shape[0]
    B = batch * seq
    idx = tokens.reshape(-1).astype(jnp.int32)
    Wb = W.astype(jnp.bfloat16)
    b2 = b.reshape(1, M)

    n_chunks = 1
    Bc = B // n_chunks
    embs = [_sc_gather(table, lax.slice(idx, (c * Bc,), ((c + 1) * Bc,)))
            for c in range(n_chunks)]
    out = None
    for c in range(n_chunks):
        out = _tc_project_chunk(embs[c], Wb, b2, position, B, c, n_chunks, out)
    return out.reshape(batch, seq, M)


# speedup vs baseline: 1.1758x; 1.0416x over previous
import functools

import jax
import jax.numpy as jnp
from jax import lax
from jax.experimental import pallas as pl
from jax.experimental.pallas import tpu as pltpu
from jax.experimental.pallas import tpu_sc as plsc

NC = 2
NS = 16
NW = NC * NS


def _sc_gather(table, idx):
    V, D = table.shape
    B = idx.shape[0]
    b_per_w = B // NW
    CH = 128
    n_ch = b_per_w // CH
    mesh = plsc.VectorSubcoreMesh(core_axis_name="c", subcore_axis_name="s")

    @functools.partial(
        pl.kernel,
        mesh=mesh,
        out_type=jax.ShapeDtypeStruct((B, D), jnp.float32),
        scratch_types=[
            pltpu.VMEM((b_per_w,), jnp.int32),
            pltpu.VMEM((CH, D), jnp.float32),
            pltpu.SemaphoreType.DMA,
        ],
    )
    def gather_kernel(table_hbm, idx_hbm, out_hbm, idx_v, rows_v, sem):
        wid = lax.axis_index("s") * NC + lax.axis_index("c")
        base = wid * b_per_w
        pltpu.sync_copy(idx_hbm.at[pl.ds(base, b_per_w)], idx_v)

        @pl.loop(0, n_ch)
        def _(c):
            off = c * CH
            pltpu.async_copy(
                table_hbm.at[idx_v.at[pl.ds(off, CH)]], rows_v, sem
            ).wait()
            pltpu.sync_copy(rows_v, out_hbm.at[pl.ds(base + off, CH)])

    return gather_kernel(table, idx)


def _tc_project_chunk(emb, Wb, b2, position, B, chunk, n_chunks, dest):
    Bc, D = emb.shape
    M = Wb.shape[0]
    L = position.shape[0]
    BLK = 2048
    j_blocks = L // BLK
    k_blocks = Bc // L
    blk_off = chunk * (Bc // BLK)

    def mm_kernel(*refs):
        emb_ref, w_ref, b_ref, pos_ref, out_ref = refs[-5:]
        acc = lax.dot_general(
            emb_ref[...].astype(jnp.bfloat16), w_ref[...],
            dimension_numbers=(((1,), (1,)), ((), ())),
            preferred_element_type=jnp.float32,
        )
        out_ref[...] = acc + b_ref[...] + pos_ref[...]

    data_specs = [
        pl.BlockSpec((BLK, D), lambda j, k: (k * j_blocks + j, 0)),
        pl.BlockSpec((M, D), lambda j, k: (0, 0)),
        pl.BlockSpec((1, M), lambda j, k: (0, 0)),
        pl.BlockSpec((BLK, M), lambda j, k: (j, 0)),
    ]
    if dest is None:
        in_specs, aliases, args = data_specs, {}, (emb, Wb, b2, position)
    else:
        in_specs = [pl.BlockSpec(memory_space=pl.ANY)] + data_specs
        aliases = {0: 0}
        args = (dest, emb, Wb, b2, position)

    return pl.pallas_call(
        mm_kernel,
        grid=(j_blocks, k_blocks),
        in_specs=in_specs,
        out_specs=pl.BlockSpec(
            (BLK, M), lambda j, k: (blk_off + k * j_blocks + j, 0)
        ),
        out_shape=jax.ShapeDtypeStruct((B, M), jnp.float32),
        input_output_aliases=aliases,
    )(*args)


def kernel(tokens, table, W, b, position):
    batch, seq = tokens.shape
    M = W.shape[0]
    B = batch * seq
    idx = tokens.reshape(-1).astype(jnp.int32)
    Wb = W.astype(jnp.bfloat16)
    b2 = b.reshape(1, M)

    n_chunks = 1
    Bc = B // n_chunks
    embs = [_sc_gather(table, lax.slice(idx, (c * Bc,), ((c + 1) * Bc,)))
            for c in range(n_chunks)]
    out = None
    for c in range(n_chunks):
        out = _tc_project_chunk(embs[c], Wb, b2, position, B, c, n_chunks, out)
    return out.reshape(batch, seq, M)
